# Initial kernel scaffold; baseline (speedup 1.0000x reference)
import jax
import jax.numpy as jnp
from jax.experimental import pallas as pl


def kernel(inputs, w, indices):
    raise NotImplementedError("write your pallas kernel here")


# speedup vs baseline: 6.1897x; 6.1897x over previous
import functools

import jax
import jax.numpy as jnp
from jax import lax
from jax.experimental import pallas as pl
from jax.experimental.pallas import tpu as pltpu
from jax.experimental.pallas import tpu_sc as plsc

N = 16384
B = 64
BH = B // 2
NC = 2
NT = 16
LANES = 16

STREAM = 128
NSTREAM = 8
CH = STREAM * NSTREAM
CHUNKS = 17
KT = CH * CHUNKS
K_TOTAL = KT * NT
KROWS = K_TOTAL // STREAM
ROWS_PER_TILE = N // NT


def _sc_body(x_hbm, rows_hbm, cols_hbm, w_hbm, zeros_hbm, out_hbm,
             idx_r, idx_c, w_v, gath, acc, gsem, ssem):
    c = lax.axis_index("c")
    s = lax.axis_index("s")

    pltpu.sync_copy(zeros_hbm.at[pl.ds(s * ROWS_PER_TILE, ROWS_PER_TILE)],
                    acc.at[pl.ds(s * ROWS_PER_TILE, ROWS_PER_TILE)])
    plsc.subcore_barrier()

    def chunk_body(ch, carry):
        rowbase = s * (CHUNKS * NSTREAM) + ch * NSTREAM
        pltpu.sync_copy(rows_hbm.at[c, pl.ds(rowbase, NSTREAM)], idx_r)
        pltpu.sync_copy(cols_hbm.at[pl.ds(rowbase, NSTREAM)], idx_c)
        pltpu.sync_copy(w_hbm.at[pl.ds(rowbase, NSTREAM)], w_v)

        copies = []
        for j in range(NSTREAM):
            copies.append(pltpu.async_copy(
                x_hbm.at[idx_r.at[j]],
                gath.at[pl.ds(j * STREAM, STREAM)], gsem))
        for cp in copies:
            cp.wait()

        for j in range(NSTREAM):
            def grp(gg, carry2, j=j):
                w16 = w_v[j, pl.ds(gg * LANES, LANES)]
                base = j * STREAM + gg * LANES
                for k in range(LANES):
                    wk = w16[k]
                    r = base + k
                    gath[r, pl.ds(0, LANES)] = gath[r, pl.ds(0, LANES)] * wk
                    gath[r, pl.ds(LANES, LANES)] = (
                        gath[r, pl.ds(LANES, LANES)] * wk)
                return carry2
            lax.fori_loop(0, STREAM // LANES, grp, 0)

        adds = []
        for j in range(NSTREAM):
            adds.append(pltpu.async_copy(
                gath.at[pl.ds(j * STREAM, STREAM)],
                acc.at[idx_c.at[j]], ssem, add=True))
        for cp in adds:
            cp.wait()
        return carry

    lax.fori_loop(0, CHUNKS, chunk_body, 0)

    plsc.subcore_barrier()
    pltpu.sync_copy(acc.at[pl.ds(s * ROWS_PER_TILE, ROWS_PER_TILE)],
                    out_hbm.at[c, pl.ds(s * ROWS_PER_TILE, ROWS_PER_TILE)])


@jax.jit
def _sparse_matmul(xstack, rows2, cols_r, w_r, zeros):
    mesh = plsc.VectorSubcoreMesh(core_axis_name="c", subcore_axis_name="s",
                                  num_cores=NC, num_subcores=NT)
    run = pl.kernel(
        _sc_body,
        out_type=jax.ShapeDtypeStruct((NC, N, BH), jnp.float32),
        mesh=mesh,
        scratch_types=[
            pltpu.VMEM((NSTREAM, STREAM), jnp.int32),
            pltpu.VMEM((NSTREAM, STREAM), jnp.int32),
            pltpu.VMEM((NSTREAM, STREAM), jnp.float32),
            pltpu.VMEM((CH, BH), jnp.float32),
            pltpu.VMEM_SHARED((N, BH), jnp.float32),
            pltpu.SemaphoreType.DMA,
            pltpu.SemaphoreType.DMA,
        ],
        compiler_params=pltpu.CompilerParams(use_tc_tiling_on_sc=False),
    )
    return run(xstack, rows2, cols_r, w_r, zeros)


def kernel(inputs, w, indices):
    nnz = indices.shape[0]
    rows = indices[:, 0].astype(jnp.int32)
    cols = indices[:, 1].astype(jnp.int32)

    pad = K_TOTAL - nnz
    rows = jnp.pad(rows, (0, pad))
    cols = jnp.pad(cols, (0, pad))
    wp = jnp.pad(w.astype(jnp.float32), (0, pad))

    xT = inputs.astype(jnp.float32).T
    xstack = jnp.concatenate([xT[:, :BH], xT[:, BH:]], axis=0)

    rows2 = jnp.stack([rows, rows + N]).reshape(NC, KROWS, STREAM)
    cols_r = cols.reshape(KROWS, STREAM)
    w_r = wp.reshape(KROWS, STREAM)
    zeros = jnp.zeros((N, BH), jnp.float32)

    o = _sparse_matmul(xstack, rows2, cols_r, w_r, zeros)
    return jnp.concatenate([o[0], o[1]], axis=1).T
